# baseline (speedup 1.0000x reference)
import functools

import jax
import jax.numpy as jnp
from jax import lax
from jax.experimental import pallas as pl
from jax.experimental.pallas import tpu as pltpu
from jax.experimental.pallas import tpu_sc as plsc

C = 1024
WN = 128
NSTAT = 10000
NPAD = 10240
IDX_CHUNK = 128


def _sc_gather(table, idx, s_pad):
  info = plsc.get_sparse_core_info()
  nworkers = info.num_cores * info.num_subcores
  per_w = s_pad // nworkers
  n_chunks = per_w // IDX_CHUNK
  mesh = plsc.VectorSubcoreMesh(core_axis_name="c", subcore_axis_name="s")

  @functools.partial(
      pl.kernel,
      mesh=mesh,
      compiler_params=pltpu.CompilerParams(use_tc_tiling_on_sc=False),
      out_type=jax.ShapeDtypeStruct((s_pad, 8), jnp.int32),
      scratch_types=[
          pltpu.VMEM((IDX_CHUNK,), jnp.int32),
          pltpu.VMEM((IDX_CHUNK, 8), jnp.int32),
          pltpu.SemaphoreType.DMA,
      ],
  )
  def gather_kernel(table_hbm, idx_hbm, out_hbm, idx_v, rows_v, sem):
    wid = lax.axis_index("s") * info.num_cores + lax.axis_index("c")
    base = wid * per_w

    def body(i, carry):
      off = base + i * IDX_CHUNK
      pltpu.sync_copy(idx_hbm.at[pl.ds(off, IDX_CHUNK)], idx_v)
      pltpu.async_copy(table_hbm.at[idx_v], rows_v, sem).wait()
      pltpu.sync_copy(rows_v, out_hbm.at[pl.ds(off, IDX_CHUNK)])
      return carry

    lax.fori_loop(0, n_chunks, body, 0)

  return gather_kernel(table, idx)


def _popcount32(x):
  x = x - (lax.shift_right_logical(x, 1) & 0x55555555)
  x = (x & 0x33333333) + (lax.shift_right_logical(x, 2) & 0x33333333)
  x = (x + lax.shift_right_logical(x, 4)) & 0x0F0F0F0F
  return lax.shift_right_logical(x * 0x01010101, 24)


def _attn_body(q_smem, seg_ref, words_ref, out_ref, l_ref, *, nchunks, nb):
  c = pl.program_id(0)

  @pl.when(c == 0)
  def _init():
    l_ref[...] = jnp.zeros_like(l_ref)
    out_ref[...] = jnp.zeros_like(out_ref)

  seg = seg_ref[0, 0, :]
  words = words_ref[...]

  es_rows = []
  for b in range(nb):
    match = _popcount32(words[0, :] & q_smem[b, 0])
    okc = _popcount32(words[4, :] & q_smem[b, 4])
    for j in range(1, 4):
      match = match + _popcount32(words[j, :] & q_smem[b, j])
      okc = okc + _popcount32(words[4 + j, :] & q_smem[b, 4 + j])
    sim = (match.astype(jnp.float32) / okc.astype(jnp.float32)).reshape(1, C)
    es_rows.append(jnp.exp(sim - 64.0))

  sub = lax.broadcasted_iota(jnp.int32, (32, C), 0)
  shift = 8 * (sub // 8) + 7 - (sub % 8)
  bits_list = []
  for j in range(4):
    w = words[j, :].reshape(1, C)
    bits_list.append(lax.shift_right_logical(w, shift) & 1)
  bits_t = jnp.concatenate(bits_list, axis=0).astype(jnp.float32)

  sb_all = jnp.concatenate(
      [bits_t * es_rows[b] for b in range(nb)], axis=0)
  es_st = jnp.concatenate(es_rows, axis=0)

  seg1 = seg.reshape(1, C)
  n_lo = jnp.min(seg)
  n_hi = jnp.max(seg)
  w0 = n_lo // WN
  nwin = n_hi // WN - w0 + 1

  def win_body(w, carry):
    nw = (w0 + w) * WN
    win_rows = nw + lax.broadcasted_iota(jnp.int32, (WN, C), 0)
    oht = (win_rows == seg1).astype(jnp.float32)
    csums = lax.dot_general(
        oht, es_st, (((1,), (1,)), ((), ())),
        preferred_element_type=jnp.float32)
    l_ref[pl.ds(nw, WN), 0:nb] = l_ref[pl.ds(nw, WN), 0:nb] + csums
    contrib = lax.dot_general(
        oht, sb_all, (((1,), (1,)), ((), ())),
        preferred_element_type=jnp.float32)
    for b in range(nb):
      out_ref[b, pl.ds(nw, WN), :] = (
          out_ref[b, pl.ds(nw, WN), :]
          + contrib[:, b * 128:(b + 1) * 128])
    return carry

  lax.fori_loop(0, nwin, win_body, 0)

  @pl.when(c == nchunks - 1)
  def _finalize():
    l = l_ref[...]
    l_safe = jnp.where(l == 0.0, 1.0, l)
    for b in range(nb):
      out_ref[b] = out_ref[b] / l_safe[:, b:b + 1]


def _attn_tc(q_all, seg_r, words, s_pad):
  nchunks = s_pad // C
  nb = q_all.shape[0]
  return pl.pallas_call(
      functools.partial(_attn_body, nchunks=nchunks, nb=nb),
      grid=(nchunks,),
      in_specs=[
          pl.BlockSpec(memory_space=pltpu.SMEM),
          pl.BlockSpec((1, 1, C), lambda c: (c, 0, 0)),
          pl.BlockSpec((8, C), lambda c: (0, c)),
      ],
      out_specs=pl.BlockSpec((nb, NPAD, 128), lambda c: (0, 0, 0)),
      out_shape=jax.ShapeDtypeStruct((nb, NPAD, 128), jnp.float32),
      scratch_shapes=[
          pltpu.VMEM((NPAD, 8), jnp.float32),
      ],
  )(q_all, seg_r, words)


def _proj_body(v_ref, w_ref, bias_ref, out_ref):
  out_ref[0] = lax.dot_general(
      v_ref[0], w_ref[...], (((1,), (0,)), ((), ())),
      preferred_element_type=jnp.float32) + bias_ref[...]


def _proj_tc(v_norm, w_o, bias):
  b = v_norm.shape[0]
  return pl.pallas_call(
      _proj_body,
      grid=(b, NPAD // WN),
      in_specs=[
          pl.BlockSpec((1, WN, 128), lambda b, i: (b, i, 0)),
          pl.BlockSpec((128, 128), lambda b, i: (0, 0)),
          pl.BlockSpec((1, 128), lambda b, i: (0, 0)),
      ],
      out_specs=pl.BlockSpec((1, WN, 128), lambda b, i: (b, i, 0)),
      out_shape=jax.ShapeDtypeStruct((b, NPAD, 128), jnp.float32),
  )(v_norm, w_o, bias)


def kernel(Q, Q_ok, ok_pos, refs, ref2seg, segments, N, W_o, b_o):
  B = Q.shape[0]
  R = refs.shape[0]
  S = ref2seg.shape[0]
  segments = segments + (jnp.asarray(N) - NSTAT).astype(segments.dtype)

  refs_w = lax.bitcast_convert_type(refs.reshape(R, 4, 4), jnp.int32)
  okp_w = lax.bitcast_convert_type(ok_pos.reshape(R, 4, 4), jnp.int32)
  table = jnp.concatenate([refs_w, okp_w], axis=1)
  q_w = lax.bitcast_convert_type(Q.reshape(B, 4, 4), jnp.int32)
  qok_w = lax.bitcast_convert_type(Q_ok.reshape(B, 4, 4), jnp.int32)
  q_all = jnp.concatenate([q_w, qok_w], axis=1)

  unit = 32 * IDX_CHUNK
  s_pad = ((S + unit - 1) // unit) * unit
  s_pad = ((s_pad + C - 1) // C) * C
  idx_pad = jnp.pad(ref2seg.astype(jnp.int32), (0, s_pad - S))
  gathered = _sc_gather(table, idx_pad, s_pad)
  words = gathered.T
  seg_pad = jnp.pad(segments.astype(jnp.int32), (0, s_pad - S),
                    constant_values=NPAD - 1)
  seg_r = seg_pad.reshape(s_pad // C, 1, C)

  v_norm = _attn_tc(q_all, seg_r, words, s_pad)
  z_pad = _proj_tc(v_norm, W_o, b_o.reshape(1, 128))
  return z_pad[:, :NSTAT, :]


# speedup vs baseline: 1.0562x; 1.0562x over previous
import functools

import jax
import jax.numpy as jnp
from jax import lax
from jax.experimental import pallas as pl
from jax.experimental.pallas import tpu as pltpu
from jax.experimental.pallas import tpu_sc as plsc

C = 1024
WN = 128
NSTAT = 10000
NPAD = 10240
IDX_CHUNK = 128


def _sc_gather(table, idx, s_pad):
  info = plsc.get_sparse_core_info()
  nworkers = info.num_cores * info.num_subcores
  per_w = s_pad // nworkers
  n_chunks = per_w // IDX_CHUNK
  mesh = plsc.VectorSubcoreMesh(core_axis_name="c", subcore_axis_name="s")

  @functools.partial(
      pl.kernel,
      mesh=mesh,
      compiler_params=pltpu.CompilerParams(use_tc_tiling_on_sc=False),
      out_type=jax.ShapeDtypeStruct((s_pad, 8), jnp.int32),
      scratch_types=[
          pltpu.VMEM((IDX_CHUNK,), jnp.int32),
          pltpu.VMEM((IDX_CHUNK, 8), jnp.int32),
          pltpu.SemaphoreType.DMA,
      ],
  )
  def gather_kernel(table_hbm, idx_hbm, out_hbm, idx_v, rows_v, sem):
    wid = lax.axis_index("s") * info.num_cores + lax.axis_index("c")
    base = wid * per_w

    def body(i, carry):
      off = base + i * IDX_CHUNK
      pltpu.sync_copy(idx_hbm.at[pl.ds(off, IDX_CHUNK)], idx_v)
      pltpu.async_copy(table_hbm.at[idx_v], rows_v, sem).wait()
      pltpu.sync_copy(rows_v, out_hbm.at[pl.ds(off, IDX_CHUNK)])
      return carry

    lax.fori_loop(0, n_chunks, body, 0)

  return gather_kernel(table, idx)


def _popcount32(x):
  x = x - (lax.shift_right_logical(x, 1) & 0x55555555)
  x = (x & 0x33333333) + (lax.shift_right_logical(x, 2) & 0x33333333)
  x = (x + lax.shift_right_logical(x, 4)) & 0x0F0F0F0F
  return lax.shift_right_logical(x * 0x01010101, 24)


def _attn_body(q_smem, seg_ref, words_ref, out_ref, l_ref, *, nchunks, nb):
  c = pl.program_id(0)

  @pl.when(c == 0)
  def _init():
    l_ref[...] = jnp.zeros_like(l_ref)
    out_ref[...] = jnp.zeros_like(out_ref)

  seg = seg_ref[0, 0, :]
  words = words_ref[...]

  es_rows = []
  for b in range(nb):
    match = _popcount32(words[0, :] & q_smem[b, 0])
    okc = _popcount32(words[4, :] & q_smem[b, 4])
    for j in range(1, 4):
      match = match + _popcount32(words[j, :] & q_smem[b, j])
      okc = okc + _popcount32(words[4 + j, :] & q_smem[b, 4 + j])
    sim = (match.astype(jnp.float32) / okc.astype(jnp.float32)).reshape(1, C)
    es_rows.append(jnp.exp(sim - 64.0))

  sub = lax.broadcasted_iota(jnp.int32, (32, C), 0)
  shift = 8 * (sub // 8) + 7 - (sub % 8)
  bits_list = []
  for j in range(4):
    w = words[j, :].reshape(1, C)
    bits_list.append(lax.shift_right_logical(w, shift) & 1)
  bits_t = jnp.concatenate(bits_list, axis=0).astype(jnp.bfloat16)

  seg1 = seg.reshape(1, C)
  n_lo = jnp.min(seg)
  n_hi = jnp.max(seg)
  w0 = n_lo // WN
  nwin = n_hi // WN - w0 + 1

  def win_body(w, carry):
    nw = (w0 + w) * WN
    win_rows = nw + lax.broadcasted_iota(jnp.int32, (WN, C), 0)
    oht = (win_rows == seg1).astype(jnp.float32)
    e_parts = []
    for b in range(nb):
      e_t = oht * es_rows[b]
      l_ref[pl.ds(nw, WN), b:b + 1] = (
          l_ref[pl.ds(nw, WN), b:b + 1]
          + jnp.sum(e_t, axis=1, keepdims=True))
      e_parts.append(e_t.astype(jnp.bfloat16))
    e_all = jnp.concatenate(e_parts, axis=0)
    contrib = lax.dot_general(
        e_all, bits_t, (((1,), (1,)), ((), ())),
        preferred_element_type=jnp.float32)
    for b in range(nb):
      out_ref[b, pl.ds(nw, WN), :] = (
          out_ref[b, pl.ds(nw, WN), :] + contrib[b * WN:(b + 1) * WN, :])
    return carry

  lax.fori_loop(0, nwin, win_body, 0)

  @pl.when(c == nchunks - 1)
  def _finalize():
    l = l_ref[...]
    l_safe = jnp.where(l == 0.0, 1.0, l)
    for b in range(nb):
      out_ref[b] = out_ref[b] / l_safe[:, b:b + 1]


def _attn_tc(q_all, seg_r, words, s_pad):
  nchunks = s_pad // C
  nb = q_all.shape[0]
  return pl.pallas_call(
      functools.partial(_attn_body, nchunks=nchunks, nb=nb),
      grid=(nchunks,),
      in_specs=[
          pl.BlockSpec(memory_space=pltpu.SMEM),
          pl.BlockSpec((1, 1, C), lambda c: (c, 0, 0)),
          pl.BlockSpec((8, C), lambda c: (0, c)),
      ],
      out_specs=pl.BlockSpec((nb, NPAD, 128), lambda c: (0, 0, 0)),
      out_shape=jax.ShapeDtypeStruct((nb, NPAD, 128), jnp.float32),
      scratch_shapes=[
          pltpu.VMEM((NPAD, 8), jnp.float32),
      ],
  )(q_all, seg_r, words)


def _proj_body(v_ref, w_ref, bias_ref, out_ref):
  out_ref[0] = lax.dot_general(
      v_ref[0], w_ref[...], (((1,), (0,)), ((), ())),
      preferred_element_type=jnp.float32) + bias_ref[...]


def _proj_tc(v_norm, w_o, bias):
  b = v_norm.shape[0]
  return pl.pallas_call(
      _proj_body,
      grid=(b, NPAD // WN),
      in_specs=[
          pl.BlockSpec((1, WN, 128), lambda b, i: (b, i, 0)),
          pl.BlockSpec((128, 128), lambda b, i: (0, 0)),
          pl.BlockSpec((1, 128), lambda b, i: (0, 0)),
      ],
      out_specs=pl.BlockSpec((1, WN, 128), lambda b, i: (b, i, 0)),
      out_shape=jax.ShapeDtypeStruct((b, NPAD, 128), jnp.float32),
  )(v_norm, w_o, bias)


def kernel(Q, Q_ok, ok_pos, refs, ref2seg, segments, N, W_o, b_o):
  B = Q.shape[0]
  R = refs.shape[0]
  S = ref2seg.shape[0]
  segments = segments + (jnp.asarray(N) - NSTAT).astype(segments.dtype)

  refs_w = lax.bitcast_convert_type(refs.reshape(R, 4, 4), jnp.int32)
  okp_w = lax.bitcast_convert_type(ok_pos.reshape(R, 4, 4), jnp.int32)
  table = jnp.concatenate([refs_w, okp_w], axis=1)
  q_w = lax.bitcast_convert_type(Q.reshape(B, 4, 4), jnp.int32)
  qok_w = lax.bitcast_convert_type(Q_ok.reshape(B, 4, 4), jnp.int32)
  q_all = jnp.concatenate([q_w, qok_w], axis=1)

  unit = 32 * IDX_CHUNK
  s_pad = ((S + unit - 1) // unit) * unit
  s_pad = ((s_pad + C - 1) // C) * C
  idx_pad = jnp.pad(ref2seg.astype(jnp.int32), (0, s_pad - S))
  gathered = _sc_gather(table, idx_pad, s_pad)
  words = gathered.T
  seg_pad = jnp.pad(segments.astype(jnp.int32), (0, s_pad - S),
                    constant_values=NPAD - 1)
  seg_r = seg_pad.reshape(s_pad // C, 1, C)

  v_norm = _attn_tc(q_all, seg_r, words, s_pad)
  z_pad = _proj_tc(v_norm, W_o, b_o.reshape(1, 128))
  return z_pad[:, :NSTAT, :]
